# baseline (speedup 1.0000x reference)
import jax
import jax.numpy as jnp
from jax import lax
from jax.experimental import pallas as pl
from jax.experimental.pallas import tpu as pltpu
from jax.experimental.pallas import tpu_sc as plsc

B = 4
S = 4096
D = 1024
NW = 32
S_PER_W = S // NW
R = 32
N_CHUNK = S_PER_W // R


def _sc_body(x_hbm, emb_hbm, out_hbm, emb_v, x_v):
    wid = lax.axis_index("s") * 2 + lax.axis_index("c")
    s0 = wid * S_PER_W

    def chunk_loop(ci, _):
        base = s0 + ci * R
        pltpu.sync_copy(emb_hbm.at[pl.ds(base, R)], emb_v)

        def batch_loop(b, _):
            pltpu.sync_copy(x_hbm.at[b, pl.ds(base, R)], x_v)

            def row_loop(r, _):
                def add_loop(j, _):
                    v = emb_v[r, pl.ds(j * 16, 16)]
                    plsc.addupdate(x_v.at[r, pl.ds(j * 16, 16)], v)
                    return 0

                lax.fori_loop(0, D // 16, add_loop, 0)
                return 0

            lax.fori_loop(0, R, row_loop, 0)
            pltpu.sync_copy(x_v, out_hbm.at[b, pl.ds(base, R)])
            return 0

        lax.fori_loop(0, B, batch_loop, 0)
        return 0

    lax.fori_loop(0, N_CHUNK, chunk_loop, 0)


@jax.jit
def kernel(x, emb):
    mesh = plsc.VectorSubcoreMesh(core_axis_name="c", subcore_axis_name="s")
    k = pl.kernel(
        _sc_body,
        out_type=jax.ShapeDtypeStruct((B, S, D), jnp.float32),
        mesh=mesh,
        scratch_types=[
            pltpu.VMEM((R, D), jnp.float32),
            pltpu.VMEM((R, D), jnp.float32),
        ],
    )
    return k(x, emb)
